# baseline (speedup 1.0000x reference)
import functools

import jax
import jax.numpy as jnp
from jax import lax
from jax.experimental import pallas as pl
from jax.experimental.pallas import tpu as pltpu
from jax.experimental.pallas import tpu_sc as plsc

NUM_BINS = 100000
EMBED_DIM = 64
MIN_VAL = 0.0
MAX_VAL = 1.0
BIN_SIZE = (MAX_VAL - MIN_VAL) / (NUM_BINS - 1)

NC = 2
NS = 16
L = 16
NW = NC * NS

B, SEQ = 4096, 200
N = B * SEQ
CH = 128
NU = N // CH
PER_W = NU // NW
NB1 = B // CH
Q = EMBED_DIM // L


def _sc_body(xt_hbm, table_hbm, out_hbm, *bufs):
    wid = lax.axis_index("s") * NC + lax.axis_index("c")
    ubase = wid * PER_W

    setA = bufs[0:9]
    setB = bufs[9:18]

    iota = lax.iota(jnp.int32, L)
    k0 = (iota & 7) * 128
    c1s = [q * 2 + (iota >> 3) for q in range(Q)]

    def fire_x(j, S):
        xbuf, xsem = S[0], S[1]
        pltpu.async_copy(xt_hbm.at[pl.ds((ubase + j) * CH, CH)], xbuf, xsem)

    def wait_x(S):
        xbuf, xsem = S[0], S[1]
        pltpu.make_async_copy(xt_hbm.at[pl.ds(0, CH)], xbuf, xsem).wait()

    def prep_fire(j, S):
        xbuf, _xs, ilo, ihi, dl, rlo, rhi, gsem, _ = S
        for g in range(CH // L):
            xg = xbuf[pl.ds(g * L, L)]
            xc = jnp.minimum(jnp.maximum(xg, MIN_VAL), MAX_VAL)
            ind = xc / jnp.float32(BIN_SIZE)
            lo = jnp.minimum(ind.astype(jnp.int32), NUM_BINS - 2)
            dl[pl.ds(g * L, L)] = ind - lo.astype(jnp.float32)
            ilo[pl.ds(g * L, L)] = lo
            ihi[pl.ds(g * L, L)] = lo + 1
        pltpu.async_copy(table_hbm.at[ilo], rlo, gsem)
        pltpu.async_copy(table_hbm.at[ihi], rhi, gsem)

    def wait_g(S):
        _, _, ilo, ihi, _dl, rlo, rhi, gsem, _ = S
        pltpu.make_async_copy(table_hbm.at[ilo], rlo, gsem).wait()
        pltpu.make_async_copy(table_hbm.at[ihi], rhi, gsem).wait()

    def wait_o(S):
        ob, osem = S[8][0], S[8][1]
        pltpu.make_async_copy(
            ob, out_hbm.at[0, :, 0, :], osem
        ).wait()

    def lerp_fire_out(j, S):
        _, _, _ilo, _ihi, dl, rlo, rhi, _g, (ob, osem) = S
        u = ubase + j
        l_idx = u // NB1
        b1 = u % NB1

        @pl.loop(0, CH // 2, unroll=2)
        def _pair(i):
            e0 = 2 * i
            e1 = e0 + 1
            bv0 = lax.broadcast_in_dim(e0, (L,), ())
            bv1 = lax.broadcast_in_dim(e1, (L,), ())
            d0 = plsc.load_gather(dl, [bv0])
            d1 = plsc.load_gather(dl, [bv1])
            a = [rlo[e, pl.ds(q * L, L)] for e in (e0, e1) for q in range(Q)]
            b = [rhi[e, pl.ds(q * L, L)] for e in (e0, e1) for q in range(Q)]
            k_0 = k0 + bv0
            k_1 = k0 + bv1
            t = [bb - aa for aa, bb in zip(a, b)]
            m = [tt * (d0 if k < Q else d1) for k, tt in enumerate(t)]
            o = [aa + mm for aa, mm in zip(a, m)]
            for q in range(Q):
                plsc.store_scatter(ob, [c1s[q], k_0], o[q])
            for q in range(Q):
                plsc.store_scatter(ob, [c1s[q], k_1], o[Q + q])

        pltpu.async_copy(ob, out_hbm.at[l_idx, :, b1, :], osem)

    fire_x(0, setA)
    fire_x(1, setB)
    wait_x(setA)
    prep_fire(0, setA)
    fire_x(2, setA)
    wait_x(setB)
    prep_fire(1, setB)
    fire_x(3, setB)
    wait_g(setA)
    lerp_fire_out(0, setA)
    wait_x(setA)
    prep_fire(2, setA)
    fire_x(4, setA)
    wait_g(setB)
    lerp_fire_out(1, setB)
    wait_x(setB)
    prep_fire(3, setB)
    fire_x(5, setB)

    @pl.loop(0, (PER_W - 6) // 2)
    def _pair_loop(p):
        c0 = 2 * p + 2
        wait_g(setA)
        wait_o(setA)
        lerp_fire_out(c0, setA)
        wait_x(setA)
        prep_fire(c0 + 2, setA)
        fire_x(c0 + 4, setA)
        wait_g(setB)
        wait_o(setB)
        lerp_fire_out(c0 + 1, setB)
        wait_x(setB)
        prep_fire(c0 + 3, setB)
        fire_x(c0 + 5, setB)

    wait_g(setA)
    wait_o(setA)
    lerp_fire_out(PER_W - 4, setA)
    wait_x(setA)
    prep_fire(PER_W - 2, setA)
    wait_g(setB)
    wait_o(setB)
    lerp_fire_out(PER_W - 3, setB)
    wait_x(setB)
    prep_fire(PER_W - 1, setB)
    wait_g(setA)
    wait_o(setA)
    lerp_fire_out(PER_W - 2, setA)
    wait_g(setB)
    wait_o(setB)
    lerp_fire_out(PER_W - 1, setB)
    wait_o(setA)
    wait_o(setB)


def _buf_set():
    return [
        pltpu.VMEM((CH,), jnp.float32),
        pltpu.SemaphoreType.DMA,
        pltpu.VMEM((CH,), jnp.int32),
        pltpu.VMEM((CH,), jnp.int32),
        pltpu.VMEM((CH,), jnp.float32),
        pltpu.VMEM((CH, EMBED_DIM), jnp.float32),
        pltpu.VMEM((CH, EMBED_DIM), jnp.float32),
        pltpu.SemaphoreType.DMA,
        (
            pltpu.VMEM((EMBED_DIM // 8, 1024), jnp.float32),
            pltpu.SemaphoreType.DMA,
        ),
    ]


def _run(x, table):
    xtf = jnp.transpose(x).reshape(N)
    mesh = plsc.VectorSubcoreMesh(
        core_axis_name="c", subcore_axis_name="s", num_cores=NC, num_subcores=NS
    )
    call = pl.kernel(
        _sc_body,
        out_type=jax.ShapeDtypeStruct((SEQ, EMBED_DIM // 8, NB1, 1024), jnp.float32),
        mesh=mesh,
        compiler_params=pltpu.CompilerParams(
            needs_layout_passes=False, use_tc_tiling_on_sc=False
        ),
        scratch_types=_buf_set() + _buf_set(),
    )
    out4 = call(xtf, table)
    out5 = out4.reshape(SEQ, EMBED_DIM // 8, NB1, 8, CH)
    return jnp.transpose(out5, (2, 4, 0, 1, 3)).reshape(B, SEQ, EMBED_DIM)


kernel = jax.jit(_run)


# speedup vs baseline: 2.5657x; 2.5657x over previous
import functools

import jax
import jax.numpy as jnp
from jax import lax
from jax.experimental import pallas as pl
from jax.experimental.pallas import tpu as pltpu
from jax.experimental.pallas import tpu_sc as plsc

NUM_BINS = 100000
EMBED_DIM = 64
MIN_VAL = 0.0
MAX_VAL = 1.0
BIN_SIZE = (MAX_VAL - MIN_VAL) / (NUM_BINS - 1)

NC = 2
NS = 16
L = 16
NW = NC * NS

B, SEQ = 4096, 200
N = B * SEQ
CH = 128
NU = N // CH
PER_W = NU // NW
NB1 = B // CH
Q = EMBED_DIM // L


def _sc_body(xt_hbm, table_hbm, out_hbm, *bufs):
    wid = lax.axis_index("s") * NC + lax.axis_index("c")
    ubase = wid * PER_W

    setA = bufs[0:9]
    setB = bufs[9:18]

    iota = lax.iota(jnp.int32, L)
    c0v = iota & 7
    c1s = [q * 2 + (iota >> 3) for q in range(Q)]

    def fire_x(j, S):
        xbuf, xsem = S[0], S[1]
        pltpu.async_copy(xt_hbm.at[pl.ds((ubase + j) * CH, CH)], xbuf, xsem)

    def wait_x(S):
        xbuf, xsem = S[0], S[1]
        pltpu.make_async_copy(xt_hbm.at[pl.ds(0, CH)], xbuf, xsem).wait()

    def prep_fire(j, S):
        xbuf, _xs, ilo, ihi, dl, rlo, rhi, gsem, _ = S
        for g in range(CH // L):
            xg = xbuf[pl.ds(g * L, L)]
            xc = jnp.minimum(jnp.maximum(xg, MIN_VAL), MAX_VAL)
            ind = xc / jnp.float32(BIN_SIZE)
            lo = jnp.minimum(ind.astype(jnp.int32), NUM_BINS - 2)
            dl[pl.ds(g * L, L)] = ind - lo.astype(jnp.float32)
            ilo[pl.ds(g * L, L)] = lo
            ihi[pl.ds(g * L, L)] = lo + 1
        pltpu.async_copy(table_hbm.at[ilo], rlo, gsem)
        pltpu.async_copy(table_hbm.at[ihi], rhi, gsem)

    def wait_g(S):
        _, _, ilo, ihi, _dl, rlo, rhi, gsem, _ = S
        pltpu.make_async_copy(table_hbm.at[ilo], rlo, gsem).wait()
        pltpu.make_async_copy(table_hbm.at[ihi], rhi, gsem).wait()

    def wait_o(S):
        ob, osem = S[8][0], S[8][1]
        pltpu.make_async_copy(
            ob.at[:, :, pl.ds(0, CH)], out_hbm.at[0, :, 0, :, :], osem
        ).wait()

    def lerp_fire_out(j, S):
        _, _, _ilo, _ihi, dl, rlo, rhi, _g, (ob, osem) = S
        u = ubase + j
        l_idx = u // NB1
        b1 = u % NB1

        @pl.loop(0, CH // 2, unroll=2)
        def _pair(i):
            e0 = 2 * i
            e1 = e0 + 1
            bv0 = lax.broadcast_in_dim(e0, (L,), ())
            bv1 = lax.broadcast_in_dim(e1, (L,), ())
            d0 = plsc.load_gather(dl, [bv0])
            d1 = plsc.load_gather(dl, [bv1])
            a = [rlo[e, pl.ds(q * L, L)] for e in (e0, e1) for q in range(Q)]
            b = [rhi[e, pl.ds(q * L, L)] for e in (e0, e1) for q in range(Q)]
            t = [bb - aa for aa, bb in zip(a, b)]
            m = [tt * (d0 if k < Q else d1) for k, tt in enumerate(t)]
            o = [aa + mm for aa, mm in zip(a, m)]
            for q in range(Q):
                plsc.store_scatter(ob, [c1s[q], c0v, bv0], o[q])
            for q in range(Q):
                plsc.store_scatter(ob, [c1s[q], c0v, bv1], o[Q + q])

        pltpu.async_copy(
            ob.at[:, :, pl.ds(0, CH)], out_hbm.at[l_idx, :, b1, :, :], osem
        )

    fire_x(0, setA)
    fire_x(1, setB)
    wait_x(setA)
    prep_fire(0, setA)
    fire_x(2, setA)
    wait_x(setB)
    prep_fire(1, setB)
    fire_x(3, setB)
    wait_g(setA)
    lerp_fire_out(0, setA)
    wait_x(setA)
    prep_fire(2, setA)
    fire_x(4, setA)
    wait_g(setB)
    lerp_fire_out(1, setB)
    wait_x(setB)
    prep_fire(3, setB)
    fire_x(5, setB)

    @pl.loop(0, (PER_W - 6) // 2)
    def _pair_loop(p):
        c0 = 2 * p + 2
        wait_g(setA)
        wait_o(setA)
        lerp_fire_out(c0, setA)
        wait_x(setA)
        prep_fire(c0 + 2, setA)
        fire_x(c0 + 4, setA)
        wait_g(setB)
        wait_o(setB)
        lerp_fire_out(c0 + 1, setB)
        wait_x(setB)
        prep_fire(c0 + 3, setB)
        fire_x(c0 + 5, setB)

    wait_g(setA)
    wait_o(setA)
    lerp_fire_out(PER_W - 4, setA)
    wait_x(setA)
    prep_fire(PER_W - 2, setA)
    wait_g(setB)
    wait_o(setB)
    lerp_fire_out(PER_W - 3, setB)
    wait_x(setB)
    prep_fire(PER_W - 1, setB)
    wait_g(setA)
    wait_o(setA)
    lerp_fire_out(PER_W - 2, setA)
    wait_g(setB)
    wait_o(setB)
    lerp_fire_out(PER_W - 1, setB)
    wait_o(setA)
    wait_o(setB)


def _buf_set():
    return [
        pltpu.VMEM((CH,), jnp.float32),
        pltpu.SemaphoreType.DMA,
        pltpu.VMEM((CH,), jnp.int32),
        pltpu.VMEM((CH,), jnp.int32),
        pltpu.VMEM((CH,), jnp.float32),
        pltpu.VMEM((CH, EMBED_DIM), jnp.float32),
        pltpu.VMEM((CH, EMBED_DIM), jnp.float32),
        pltpu.SemaphoreType.DMA,
        (
            pltpu.VMEM((EMBED_DIM // 8, 8, CH + 1), jnp.float32),
            pltpu.SemaphoreType.DMA,
        ),
    ]


def _run(x, table):
    xtf = jnp.transpose(x).reshape(N)
    mesh = plsc.VectorSubcoreMesh(
        core_axis_name="c", subcore_axis_name="s", num_cores=NC, num_subcores=NS
    )
    call = pl.kernel(
        _sc_body,
        out_type=jax.ShapeDtypeStruct(
            (SEQ, EMBED_DIM // 8, NB1, 8, CH), jnp.float32
        ),
        mesh=mesh,
        compiler_params=pltpu.CompilerParams(
            needs_layout_passes=False, use_tc_tiling_on_sc=False
        ),
        scratch_types=_buf_set() + _buf_set(),
    )
    out5 = call(xtf, table)
    return jnp.transpose(out5, (2, 4, 0, 1, 3)).reshape(B, SEQ, EMBED_DIM)


kernel = jax.jit(_run)
